# baseline (speedup 1.0000x reference)
import functools

import jax
import jax.numpy as jnp
from jax import lax
from jax.experimental import pallas as pl
from jax.experimental.pallas import tpu as pltpu
from jax.experimental.pallas import tpu_sc as plsc

T = 32768
HIDDEN = 768
E = 8
K = 2
LANES = 16
NWORKERS = 32
TOK_PER_W = T // NWORKERS
BT = 1024


NSTREAM = 4
TQ = T // NSTREAM
NB = TQ // BT


def _mm_body(*refs):
    x_refs = refs[:NSTREAM]
    wt_ref = refs[NSTREAM]
    o_refs = refs[NSTREAM + 1:]
    for x_ref, o_ref in zip(x_refs, o_refs):
        o_ref[...] = jnp.dot(x_ref[...], wt_ref[...],
                             preferred_element_type=jnp.float32)


def _tc_logits(x, wt):
    outs = pl.pallas_call(
        _mm_body,
        grid=(NB,),
        in_specs=[
            pl.BlockSpec((BT, HIDDEN), lambda i, j=j: (i + j * NB, 0))
            for j in range(NSTREAM)
        ] + [pl.BlockSpec((HIDDEN, E), lambda i: (0, 0))],
        out_specs=[
            pl.BlockSpec((BT, E), lambda i: (i, 0)) for _ in range(NSTREAM)
        ],
        out_shape=[
            jax.ShapeDtypeStruct((TQ, E), jnp.float32) for _ in range(NSTREAM)
        ],
    )(*([x] * NSTREAM), wt)
    return jnp.concatenate(outs, axis=0)


def _router_body(logits_hbm, scores_hbm, w_hbm, i_hbm, lg_v, sc_v, w_v, i_v):
    wid = lax.axis_index("s") * 2 + lax.axis_index("c")
    tok0 = wid * TOK_PER_W
    pltpu.sync_copy(logits_hbm.at[pl.ds(tok0, TOK_PER_W), pl.ds(0, E)], lg_v)

    lane = lax.iota(jnp.int32, 16)
    big = jnp.full((LANES,), E, jnp.int32)
    neg = jnp.full((LANES,), -3.0e38, jnp.float32)

    def body(g, carry):
        row = g * LANES + lane
        vs = [plsc.load_gather(lg_v, [row, jnp.full((LANES,), e, jnp.int32)])
              for e in range(E)]
        m = vs[0]
        for e in range(1, E):
            m = jnp.maximum(m, vs[e])
        es = [jnp.exp(v - m) for v in vs]
        s = es[0]
        for e in range(1, E):
            s = s + es[e]
        ss = [ev / s for ev in es]
        v1 = ss[0]
        for e in range(1, E):
            v1 = jnp.maximum(v1, ss[e])
        i1 = big
        for e in range(E):
            i1 = jnp.minimum(i1, jnp.where(ss[e] == v1,
                                           jnp.full((LANES,), e, jnp.int32),
                                           big))
        v2 = neg
        for e in range(E):
            ecur = jnp.full((LANES,), e, jnp.int32)
            v2 = jnp.maximum(v2, jnp.where(i1 == ecur, neg, ss[e]))
        i2 = big
        for e in range(E):
            ecur = jnp.full((LANES,), e, jnp.int32)
            i2 = jnp.minimum(i2, jnp.where((ss[e] == v2) & (i1 != ecur),
                                           ecur, big))
        for e in range(E):
            plsc.store_scatter(sc_v, [row, jnp.full((LANES,), e, jnp.int32)],
                               ss[e])
        z = jnp.zeros((LANES,), jnp.int32)
        plsc.store_scatter(w_v, [row, z], v1)
        plsc.store_scatter(w_v, [row, z + 1], v2)
        plsc.store_scatter(i_v, [row, z], i1)
        plsc.store_scatter(i_v, [row, z + 1], i2)
        return carry

    lax.fori_loop(0, 1, body, 0)

    pltpu.sync_copy(sc_v, scores_hbm.at[pl.ds(tok0, TOK_PER_W), :])
    pltpu.sync_copy(w_v, w_hbm.at[pl.ds(tok0, TOK_PER_W), :])
    pltpu.sync_copy(i_v, i_hbm.at[pl.ds(tok0, TOK_PER_W), :])


_sc_router = functools.partial(
    pl.kernel,
    out_type=(
        jax.ShapeDtypeStruct((T, E), jnp.float32),
        jax.ShapeDtypeStruct((T, K), jnp.float32),
        jax.ShapeDtypeStruct((T, K), jnp.int32),
    ),
    mesh=plsc.VectorSubcoreMesh(core_axis_name="c", subcore_axis_name="s",
                                num_cores=2, num_subcores=16),
    scratch_types=[
        pltpu.VMEM((TOK_PER_W, E), jnp.float32),
        pltpu.VMEM((TOK_PER_W, E), jnp.float32),
        pltpu.VMEM((TOK_PER_W, K), jnp.float32),
        pltpu.VMEM((TOK_PER_W, K), jnp.int32),
    ],
    compiler_params=pltpu.CompilerParams(needs_layout_passes=False,
                                         use_tc_tiling_on_sc=False),
)(_router_body)


@jax.jit
def kernel(x, W):
    logits = _tc_logits(x, W.T)
    scores, expert_weights, expert_indices = _sc_router(x)
    return (scores, logits, expert_weights, expert_indices)


# speedup vs baseline: 5.3825x; 5.3825x over previous
import jax
import jax.numpy as jnp
from jax import lax
from jax.experimental import pallas as pl

T = 32768
HIDDEN = 768
E = 8
K = 2
BT = 1024
NSTREAM = 4
TQ = T // NSTREAM
NB = TQ // BT


def _router_block(w_ref, x_ref, lt_ref, st_ref, wt_ref, it_ref):
    lt = lax.dot_general(w_ref[...], x_ref[...],
                         (((1,), (1,)), ((), ())),
                         preferred_element_type=jnp.float32)
    lt_ref[...] = lt
    rows = [lt[e, :] for e in range(E)]
    m = rows[0]
    for e in range(1, E):
        m = jnp.maximum(m, rows[e])
    ex = [jnp.exp(r - m) for r in rows]
    s = ex[0]
    for e in range(1, E):
        s = s + ex[e]
    inv = 1.0 / s
    sc = [ev * inv for ev in ex]
    for e in range(E):
        st_ref[e, :] = sc[e]
    v1 = sc[0]
    for e in range(1, E):
        v1 = jnp.maximum(v1, sc[e])
    big = jnp.full((BT,), E, jnp.int32)
    i1 = big
    for e in range(E):
        i1 = jnp.minimum(i1, jnp.where(sc[e] == v1,
                                       jnp.full((BT,), e, jnp.int32), big))
    neg = jnp.full((BT,), -3.0e38, jnp.float32)
    v2 = neg
    for e in range(E):
        ecur = jnp.full((BT,), e, jnp.int32)
        v2 = jnp.maximum(v2, jnp.where(i1 == ecur, neg, sc[e]))
    i2 = big
    for e in range(E):
        ecur = jnp.full((BT,), e, jnp.int32)
        i2 = jnp.minimum(i2, jnp.where((sc[e] == v2) & (i1 != ecur),
                                       ecur, big))
    wt_ref[0, :] = v1
    wt_ref[1, :] = v2
    it_ref[0, :] = i1
    it_ref[1, :] = i2


def _body(*refs):
    w_ref = refs[0]
    x_refs = refs[1:1 + NSTREAM]
    out_refs = refs[1 + NSTREAM:]
    for j in range(NSTREAM):
        _router_block(w_ref, x_refs[j], out_refs[4 * j], out_refs[4 * j + 1],
                      out_refs[4 * j + 2], out_refs[4 * j + 3])


def _fused_router(w, x):
    out_specs = []
    out_shape = []
    for _ in range(NSTREAM):
        out_specs += [
            pl.BlockSpec((E, BT), lambda i: (0, i)),
            pl.BlockSpec((E, BT), lambda i: (0, i)),
            pl.BlockSpec((K, BT), lambda i: (0, i)),
            pl.BlockSpec((K, BT), lambda i: (0, i)),
        ]
        out_shape += [
            jax.ShapeDtypeStruct((E, TQ), jnp.float32),
            jax.ShapeDtypeStruct((E, TQ), jnp.float32),
            jax.ShapeDtypeStruct((K, TQ), jnp.float32),
            jax.ShapeDtypeStruct((K, TQ), jnp.int32),
        ]
    return pl.pallas_call(
        _body,
        grid=(NB,),
        in_specs=[pl.BlockSpec((E, HIDDEN), lambda i: (0, 0))] + [
            pl.BlockSpec((BT, HIDDEN), lambda i, j=j: (i + j * NB, 0))
            for j in range(NSTREAM)
        ],
        out_specs=out_specs,
        out_shape=out_shape,
    )(w, *([x] * NSTREAM))


@jax.jit
def kernel(x, W):
    outs = _fused_router(W, x)
    lts = [outs[4 * j] for j in range(NSTREAM)]
    sts = [outs[4 * j + 1] for j in range(NSTREAM)]
    wts = [outs[4 * j + 2] for j in range(NSTREAM)]
    its = [outs[4 * j + 3] for j in range(NSTREAM)]
    logits = jnp.concatenate(lts, axis=1).T
    scores = jnp.concatenate(sts, axis=1).T
    expert_weights = jnp.concatenate(wts, axis=1).T
    expert_indices = jnp.concatenate(its, axis=1).T
    return (scores, logits, expert_weights, expert_indices)
